# Initial kernel scaffold; baseline (speedup 1.0000x reference)
import jax
import jax.numpy as jnp
from jax.experimental import pallas as pl


def kernel(seeds, nbr_ev, event_src, event_dst, event_ts_s, event_w, src_emb, dst_emb, W1, b1, W2, b2, W_event_src, W_event_dst, W_self, W_neigh):
    raise NotImplementedError("write your pallas kernel here")


# speedup vs baseline: 2.5887x; 2.5887x over previous
import functools

import jax
import jax.numpy as jnp
from jax import lax
from jax.experimental import pallas as pl
from jax.experimental.pallas import tpu as pltpu
from jax.experimental.pallas import tpu_sc as plsc

B = 16384
FANOUT = 16
EV = B * FANOUT
N = 100000
D = 128
H = 128
TS_RANGE = 86400.0

NC = 2
NS = 16
NW = NC * NS
EV_PER_W = EV // NW
CH = 256
NCHUNK = EV_PER_W // CH
SEEDS_PER_W = B // NW


def _precompute_body(src_ref, dst_ref, wes_ref, wed_ref, ps_ref, pd_ref):
    dn = (((1,), (1,)), ((), ()))
    ps_ref[...] = lax.dot_general(src_ref[...], wes_ref[...], dn,
                                  preferred_element_type=jnp.float32)
    pd_ref[...] = lax.dot_general(dst_ref[...], wed_ref[...], dn,
                                  preferred_element_type=jnp.float32)


def _precompute_tables(src_emb, dst_emb, wes, wed):
    rb = 1024
    nblk = (N + rb - 1) // rb
    return pl.pallas_call(
        _precompute_body,
        grid=(nblk,),
        in_specs=[
            pl.BlockSpec((rb, D), lambda i: (i, 0)),
            pl.BlockSpec((rb, D), lambda i: (i, 0)),
            pl.BlockSpec((H, D), lambda i: (0, 0)),
            pl.BlockSpec((H, D), lambda i: (0, 0)),
        ],
        out_specs=[
            pl.BlockSpec((rb, H), lambda i: (i, 0)),
            pl.BlockSpec((rb, H), lambda i: (i, 0)),
        ],
        out_shape=[
            jax.ShapeDtypeStruct((N, H), jnp.float32),
            jax.ShapeDtypeStruct((N, H), jnp.float32),
        ],
    )(src_emb, dst_emb, wes, wed)


def _sc_body(ev_hbm, seeds_hbm, esrc_hbm, edst_hbm, ets_hbm, ew_hbm,
             psrc_hbm, pdst_hbm, semb_hbm,
             g_out, ts_out, w_out, self_out,
             ev_v, sidx_v, didx_v, ts_v, w_v, rows_a, rows_b,
             sem_s, sem_a, sem_b):
    wid = lax.axis_index("c") * NS + lax.axis_index("s")
    base = wid * EV_PER_W

    def chunk(i, carry):
        off = base + i * CH
        pltpu.sync_copy(ev_hbm.at[pl.ds(off, CH)], ev_v)
        c1 = pltpu.async_copy(esrc_hbm.at[ev_v], sidx_v, sem_s)
        c2 = pltpu.async_copy(edst_hbm.at[ev_v], didx_v, sem_s)
        c3 = pltpu.async_copy(ets_hbm.at[ev_v], ts_v, sem_s)
        c4 = pltpu.async_copy(ew_hbm.at[ev_v], w_v, sem_s)
        c1.wait()
        c2.wait()
        ca = pltpu.async_copy(psrc_hbm.at[sidx_v], rows_a, sem_a)
        cb = pltpu.async_copy(pdst_hbm.at[didx_v], rows_b, sem_b)
        c3.wait()
        c4.wait()
        pltpu.sync_copy(ts_v, ts_out.at[pl.ds(off, CH)])
        pltpu.sync_copy(w_v, w_out.at[pl.ds(off, CH)])
        ca.wait()
        cb.wait()

        def addrow(r, carry2):
            for c in range(H // 16):
                sl = pl.ds(c * 16, 16)
                rows_a[r, sl] = rows_a[r, sl] + rows_b[r, sl]
            return carry2

        lax.fori_loop(0, CH, addrow, 0, unroll=2)
        pltpu.sync_copy(rows_a, g_out.at[pl.ds(off, CH)])
        return carry

    lax.fori_loop(0, NCHUNK, chunk, 0)

    sbase = wid * SEEDS_PER_W
    for j in range(SEEDS_PER_W // CH):
        soff = sbase + j * CH
        pltpu.sync_copy(seeds_hbm.at[pl.ds(soff, CH)], ev_v)
        pltpu.async_copy(semb_hbm.at[ev_v], rows_a, sem_a).wait()
        pltpu.sync_copy(rows_a, self_out.at[pl.ds(soff, CH)])


def _sc_gather(ev, seeds, esrc, edst, ets, ew, psrc, pdst, semb):
    mesh = plsc.VectorSubcoreMesh(core_axis_name="c", subcore_axis_name="s")
    fn = pl.kernel(
        _sc_body,
        out_type=[
            jax.ShapeDtypeStruct((EV, H), jnp.float32),
            jax.ShapeDtypeStruct((EV,), jnp.float32),
            jax.ShapeDtypeStruct((EV,), jnp.float32),
            jax.ShapeDtypeStruct((B, D), jnp.float32),
        ],
        mesh=mesh,
        scratch_types=[
            pltpu.VMEM((CH,), jnp.int32),
            pltpu.VMEM((CH,), jnp.int32),
            pltpu.VMEM((CH,), jnp.int32),
            pltpu.VMEM((CH,), jnp.float32),
            pltpu.VMEM((CH,), jnp.float32),
            pltpu.VMEM((CH, H), jnp.float32),
            pltpu.VMEM((CH, H), jnp.float32),
            pltpu.SemaphoreType.DMA,
            pltpu.SemaphoreType.DMA,
            pltpu.SemaphoreType.DMA,
        ],
    )
    return fn(ev, seeds, esrc, edst, ets, ew, psrc, pdst, semb)


def _finish_body(g_ref, ts_ref, w_ref, self_ref, w1t_ref, b1_ref, w2_ref,
                 b2_ref, wself_ref, wneigh_ref, out_ref):
    dn = (((1,), (1,)), ((), ()))
    ts_n = ts_ref[...] * (1.0 / TS_RANGE)
    wv = jnp.log1p(w_ref[...])
    h1 = jnp.maximum(
        ts_n * w1t_ref[0:1, :] + wv * w1t_ref[1:2, :] + b1_ref[...], 0.0)
    mlp = lax.dot_general(h1, w2_ref[...], dn,
                          preferred_element_type=jnp.float32) + b2_ref[...]
    evh = jnp.maximum(mlp + g_ref[...], 0.0)
    sb = evh.shape[0] // FANOUT
    neigh = jnp.mean(evh.reshape(sb, FANOUT, H), axis=1)
    out = lax.dot_general(self_ref[...], wself_ref[...], dn,
                          preferred_element_type=jnp.float32)
    out += lax.dot_general(neigh, wneigh_ref[...], dn,
                           preferred_element_type=jnp.float32)
    out_ref[...] = jnp.maximum(out, 0.0)


def _finish(g, ts, w, self_rows, w1t, b1, w2, b2, wself, wneigh):
    sb = 256
    eb = sb * FANOUT
    nblk = B // sb
    return pl.pallas_call(
        _finish_body,
        grid=(nblk,),
        in_specs=[
            pl.BlockSpec((eb, H), lambda i: (i, 0)),
            pl.BlockSpec((eb, 1), lambda i: (i, 0)),
            pl.BlockSpec((eb, 1), lambda i: (i, 0)),
            pl.BlockSpec((sb, D), lambda i: (i, 0)),
            pl.BlockSpec((2, H), lambda i: (0, 0)),
            pl.BlockSpec((1, H), lambda i: (0, 0)),
            pl.BlockSpec((H, H), lambda i: (0, 0)),
            pl.BlockSpec((1, H), lambda i: (0, 0)),
            pl.BlockSpec((H, D), lambda i: (0, 0)),
            pl.BlockSpec((H, H), lambda i: (0, 0)),
        ],
        out_specs=pl.BlockSpec((sb, H), lambda i: (i, 0)),
        out_shape=jax.ShapeDtypeStruct((B, H), jnp.float32),
    )(g, ts, w, self_rows, w1t, b1, w2, b2, wself, wneigh)


def kernel(seeds, nbr_ev, event_src, event_dst, event_ts_s, event_w,
           src_emb, dst_emb, W1, b1, W2, b2,
           W_event_src, W_event_dst, W_self, W_neigh):
    ev = nbr_ev.reshape(EV).astype(jnp.int32)
    seeds32 = seeds.astype(jnp.int32)
    psrc, pdst = _precompute_tables(src_emb, dst_emb, W_event_src,
                                    W_event_dst)
    g, ts_g, w_g, self_rows = _sc_gather(
        ev, seeds32, event_src, event_dst, event_ts_s, event_w,
        psrc, pdst, src_emb)
    out = _finish(g, ts_g.reshape(EV, 1), w_g.reshape(EV, 1), self_rows,
                  W1.T, b1.reshape(1, H), W2, b2.reshape(1, H),
                  W_self, W_neigh)
    return out


# speedup vs baseline: 3.0507x; 1.1785x over previous
import functools

import jax
import jax.numpy as jnp
from jax import lax
from jax.experimental import pallas as pl
from jax.experimental.pallas import tpu as pltpu
from jax.experimental.pallas import tpu_sc as plsc

B = 16384
FANOUT = 16
EV = B * FANOUT
N = 100000
D = 128
H = 128
TS_RANGE = 86400.0

NC = 2
NS = 16
NW = NC * NS
EV_PER_W = EV // NW
CH = 128
NCHUNK = EV_PER_W // CH
SEEDS_PER_W = B // NW


def _precompute_body(src_ref, dst_ref, wes_ref, wed_ref, ps_ref, pd_ref):
    dn = (((1,), (1,)), ((), ()))
    ps_ref[...] = lax.dot_general(src_ref[...], wes_ref[...], dn,
                                  preferred_element_type=jnp.float32)
    pd_ref[...] = lax.dot_general(dst_ref[...], wed_ref[...], dn,
                                  preferred_element_type=jnp.float32)


def _precompute_tables(src_emb, dst_emb, wes, wed):
    rb = 1024
    nblk = (N + rb - 1) // rb
    return pl.pallas_call(
        _precompute_body,
        grid=(nblk,),
        in_specs=[
            pl.BlockSpec((rb, D), lambda i: (i, 0)),
            pl.BlockSpec((rb, D), lambda i: (i, 0)),
            pl.BlockSpec((H, D), lambda i: (0, 0)),
            pl.BlockSpec((H, D), lambda i: (0, 0)),
        ],
        out_specs=[
            pl.BlockSpec((rb, H), lambda i: (i, 0)),
            pl.BlockSpec((rb, H), lambda i: (i, 0)),
        ],
        out_shape=[
            jax.ShapeDtypeStruct((N, H), jnp.float32),
            jax.ShapeDtypeStruct((N, H), jnp.float32),
        ],
    )(src_emb, dst_emb, wes, wed)


def _add_rows(dst, src):
    def addrow(r, carry):
        for c in range(H // 16):
            sl = pl.ds(c * 16, 16)
            dst[r, sl] = dst[r, sl] + src[r, sl]
        return carry

    lax.fori_loop(0, dst.shape[0], addrow, 0, unroll=2)


def _sc_body(ev_hbm, seeds_hbm, esrc_hbm, edst_hbm, ets_hbm, ew_hbm,
             psrc_hbm, pdst_hbm, semb_hbm,
             g_out, ts_out, w_out, self_out,
             ev_all, sidx_all, didx_all, ts_all, w_all,
             rows_a0, rows_b0, rows_a1, rows_b1,
             sem_s, sem_g0, sem_g1, sem_w):
    wid = lax.axis_index("c") * NS + lax.axis_index("s")
    base = wid * EV_PER_W

    pltpu.async_copy(ev_hbm.at[pl.ds(base, EV_PER_W)], ev_all, sem_s).wait()
    c1 = pltpu.async_copy(esrc_hbm.at[ev_all], sidx_all, sem_s)
    c2 = pltpu.async_copy(edst_hbm.at[ev_all], didx_all, sem_s)
    c3 = pltpu.async_copy(ets_hbm.at[ev_all], ts_all, sem_s)
    c4 = pltpu.async_copy(ew_hbm.at[ev_all], w_all, sem_s)
    c3.wait()
    c4.wait()
    cw1 = pltpu.async_copy(ts_all, ts_out.at[pl.ds(base, EV_PER_W)], sem_w)
    cw2 = pltpu.async_copy(w_all, w_out.at[pl.ds(base, EV_PER_W)], sem_w)
    c1.wait()
    c2.wait()

    rows = ((rows_a0, rows_b0, sem_g0), (rows_a1, rows_b1, sem_g1))

    def issue(i, slot):
        ra, rb, sg = rows[slot]
        sl = pl.ds(i * CH, CH)
        ca = pltpu.async_copy(psrc_hbm.at[sidx_all.at[sl]], ra, sg)
        cb = pltpu.async_copy(pdst_hbm.at[didx_all.at[sl]], rb, sg)
        return ca, cb

    def drain(slot):
        ra, rb, sg = rows[slot]
        pltpu.make_async_copy(psrc_hbm.at[sidx_all.at[pl.ds(0, CH)]], ra,
                              sg).wait()
        pltpu.make_async_copy(pdst_hbm.at[didx_all.at[pl.ds(0, CH)]], rb,
                              sg).wait()

    def finish(i, slot):
        ra, rb, _ = rows[slot]
        _add_rows(ra, rb)
        pltpu.sync_copy(ra, g_out.at[pl.ds(base + i * CH, CH)])

    issue(0, 0)

    def super_iter(k, carry):
        i0 = 2 * k
        drain(0)
        issue(i0 + 1, 1)
        finish(i0, 0)
        drain(1)

        @pl.when(k < NCHUNK // 2 - 1)
        def _():
            issue(i0 + 2, 0)

        finish(i0 + 1, 1)
        return carry

    lax.fori_loop(0, NCHUNK // 2, super_iter, 0)

    cw1.wait()
    cw2.wait()

    sbase = wid * SEEDS_PER_W
    for j in range(SEEDS_PER_W // CH):
        soff = sbase + j * CH
        ra, _, sg = rows[j % 2]
        pltpu.sync_copy(seeds_hbm.at[pl.ds(soff, CH)], sidx_all.at[pl.ds(0, CH)])
        pltpu.async_copy(semb_hbm.at[sidx_all.at[pl.ds(0, CH)]], ra, sg).wait()
        pltpu.sync_copy(ra, self_out.at[pl.ds(soff, CH)])


def _sc_gather(ev, seeds, esrc, edst, ets, ew, psrc, pdst, semb):
    mesh = plsc.VectorSubcoreMesh(core_axis_name="c", subcore_axis_name="s")
    fn = pl.kernel(
        _sc_body,
        out_type=[
            jax.ShapeDtypeStruct((EV, H), jnp.float32),
            jax.ShapeDtypeStruct((EV,), jnp.float32),
            jax.ShapeDtypeStruct((EV,), jnp.float32),
            jax.ShapeDtypeStruct((B, D), jnp.float32),
        ],
        mesh=mesh,
        scratch_types=[
            pltpu.VMEM((EV_PER_W,), jnp.int32),
            pltpu.VMEM((EV_PER_W,), jnp.int32),
            pltpu.VMEM((EV_PER_W,), jnp.int32),
            pltpu.VMEM((EV_PER_W,), jnp.float32),
            pltpu.VMEM((EV_PER_W,), jnp.float32),
            pltpu.VMEM((CH, H), jnp.float32),
            pltpu.VMEM((CH, H), jnp.float32),
            pltpu.VMEM((CH, H), jnp.float32),
            pltpu.VMEM((CH, H), jnp.float32),
            pltpu.SemaphoreType.DMA,
            pltpu.SemaphoreType.DMA,
            pltpu.SemaphoreType.DMA,
            pltpu.SemaphoreType.DMA,
        ],
    )
    return fn(ev, seeds, esrc, edst, ets, ew, psrc, pdst, semb)


def _finish_body(g_ref, ts_ref, w_ref, self_ref, w1t_ref, b1_ref, w2_ref,
                 b2_ref, wself_ref, wneigh_ref, out_ref):
    dn = (((1,), (1,)), ((), ()))
    ts_n = ts_ref[...] * (1.0 / TS_RANGE)
    wv = jnp.log1p(w_ref[...])
    h1 = jnp.maximum(
        ts_n * w1t_ref[0:1, :] + wv * w1t_ref[1:2, :] + b1_ref[...], 0.0)
    mlp = lax.dot_general(h1, w2_ref[...], dn,
                          preferred_element_type=jnp.float32) + b2_ref[...]
    evh = jnp.maximum(mlp + g_ref[...], 0.0)
    sb = evh.shape[0] // FANOUT
    neigh = jnp.mean(evh.reshape(sb, FANOUT, H), axis=1)
    out = lax.dot_general(self_ref[...], wself_ref[...], dn,
                          preferred_element_type=jnp.float32)
    out += lax.dot_general(neigh, wneigh_ref[...], dn,
                           preferred_element_type=jnp.float32)
    out_ref[...] = jnp.maximum(out, 0.0)


def _finish(g, ts, w, self_rows, w1t, b1, w2, b2, wself, wneigh):
    sb = 256
    eb = sb * FANOUT
    nblk = B // sb
    return pl.pallas_call(
        _finish_body,
        grid=(nblk,),
        in_specs=[
            pl.BlockSpec((eb, H), lambda i: (i, 0)),
            pl.BlockSpec((eb, 1), lambda i: (i, 0)),
            pl.BlockSpec((eb, 1), lambda i: (i, 0)),
            pl.BlockSpec((sb, D), lambda i: (i, 0)),
            pl.BlockSpec((2, H), lambda i: (0, 0)),
            pl.BlockSpec((1, H), lambda i: (0, 0)),
            pl.BlockSpec((H, H), lambda i: (0, 0)),
            pl.BlockSpec((1, H), lambda i: (0, 0)),
            pl.BlockSpec((H, D), lambda i: (0, 0)),
            pl.BlockSpec((H, H), lambda i: (0, 0)),
        ],
        out_specs=pl.BlockSpec((sb, H), lambda i: (i, 0)),
        out_shape=jax.ShapeDtypeStruct((B, H), jnp.float32),
    )(g, ts, w, self_rows, w1t, b1, w2, b2, wself, wneigh)


def kernel(seeds, nbr_ev, event_src, event_dst, event_ts_s, event_w,
           src_emb, dst_emb, W1, b1, W2, b2,
           W_event_src, W_event_dst, W_self, W_neigh):
    ev = nbr_ev.reshape(EV).astype(jnp.int32)
    seeds32 = seeds.astype(jnp.int32)
    psrc, pdst = _precompute_tables(src_emb, dst_emb, W_event_src,
                                    W_event_dst)
    g, ts_g, w_g, self_rows = _sc_gather(
        ev, seeds32, event_src, event_dst, event_ts_s, event_w,
        psrc, pdst, src_emb)
    out = _finish(g, ts_g.reshape(EV, 1), w_g.reshape(EV, 1), self_rows,
                  W1.T, b1.reshape(1, H), W2, b2.reshape(1, H),
                  W_self, W_neigh)
    return out


# speedup vs baseline: 3.8793x; 1.2716x over previous
import functools

import jax
import jax.numpy as jnp
from jax import lax
from jax.experimental import pallas as pl
from jax.experimental.pallas import tpu as pltpu
from jax.experimental.pallas import tpu_sc as plsc

B = 16384
FANOUT = 16
EV = B * FANOUT
N = 100000
D = 128
H = 128
TS_RANGE = 86400.0

NC = 2
NS = 16
NW = NC * NS
EV_PER_W = EV // NW
CH = 128
NCHUNK = EV_PER_W // CH
SEEDS_PER_W = B // NW


def _precompute_body(src_ref, dst_ref, wes_ref, wed_ref, ps_ref, pd_ref):
    dn = (((1,), (1,)), ((), ()))
    ps_ref[...] = lax.dot_general(src_ref[...], wes_ref[...], dn,
                                  preferred_element_type=jnp.float32)
    pd_ref[...] = lax.dot_general(dst_ref[...], wed_ref[...], dn,
                                  preferred_element_type=jnp.float32)


def _precompute_tables(src_emb, dst_emb, wes, wed):
    rb = 1024
    nblk = (N + rb - 1) // rb
    return pl.pallas_call(
        _precompute_body,
        grid=(nblk,),
        in_specs=[
            pl.BlockSpec((rb, D), lambda i: (i, 0)),
            pl.BlockSpec((rb, D), lambda i: (i, 0)),
            pl.BlockSpec((H, D), lambda i: (0, 0)),
            pl.BlockSpec((H, D), lambda i: (0, 0)),
        ],
        out_specs=[
            pl.BlockSpec((rb, H), lambda i: (i, 0)),
            pl.BlockSpec((rb, H), lambda i: (i, 0)),
        ],
        out_shape=[
            jax.ShapeDtypeStruct((N, H), jnp.float32),
            jax.ShapeDtypeStruct((N, H), jnp.float32),
        ],
    )(src_emb, dst_emb, wes, wed)


def _add_rows(dst, src):
    def addrow(r, carry):
        for c in range(H // 16):
            sl = pl.ds(c * 16, 16)
            dst[r, sl] = dst[r, sl] + src[r, sl]
        return carry

    lax.fori_loop(0, dst.shape[0], addrow, 0, unroll=2)


def _sc_body(ev_hbm, seeds_hbm, esrc_hbm, edst_hbm, ets_hbm, ew_hbm,
             psrc_hbm, pdst_hbm, semb_hbm,
             g_out, ts_out, w_out, self_out,
             ev_all, sidx_all, didx_all, ts_all, w_all,
             rows_a0, rows_b0, rows_a1, rows_b1,
             sem_s, sem_g0, sem_g1, sem_w):
    wid = lax.axis_index("c") * NS + lax.axis_index("s")
    base = wid * EV_PER_W

    pltpu.async_copy(ev_hbm.at[pl.ds(base, EV_PER_W)], ev_all, sem_s).wait()
    c1 = pltpu.async_copy(esrc_hbm.at[ev_all], sidx_all, sem_s)
    c2 = pltpu.async_copy(edst_hbm.at[ev_all], didx_all, sem_s)
    c3 = pltpu.async_copy(ets_hbm.at[ev_all], ts_all, sem_s)
    c4 = pltpu.async_copy(ew_hbm.at[ev_all], w_all, sem_s)
    c3.wait()
    c4.wait()
    cw1 = pltpu.async_copy(ts_all, ts_out.at[pl.ds(base, EV_PER_W)], sem_w)
    cw2 = pltpu.async_copy(w_all, w_out.at[pl.ds(base, EV_PER_W)], sem_w)
    c1.wait()
    c2.wait()

    rows = ((rows_a0, rows_b0, sem_g0), (rows_a1, rows_b1, sem_g1))

    def issue(i, slot):
        ra, rb, sg = rows[slot]
        sl = pl.ds(i * CH, CH)
        ca = pltpu.async_copy(psrc_hbm.at[sidx_all.at[sl]], ra, sg)
        cb = pltpu.async_copy(pdst_hbm.at[didx_all.at[sl]], rb, sg)
        return ca, cb

    def drain(slot):
        ra, rb, sg = rows[slot]
        pltpu.make_async_copy(psrc_hbm.at[sidx_all.at[pl.ds(0, CH)]], ra,
                              sg).wait()
        pltpu.make_async_copy(pdst_hbm.at[didx_all.at[pl.ds(0, CH)]], rb,
                              sg).wait()

    def finish(i, slot):
        ra, rb, _ = rows[slot]
        _add_rows(ra, rb)
        pltpu.sync_copy(ra, g_out.at[pl.ds(base + i * CH, CH)])

    issue(0, 0)

    def super_iter(k, carry):
        i0 = 2 * k
        drain(0)
        issue(i0 + 1, 1)
        finish(i0, 0)
        drain(1)

        @pl.when(k < NCHUNK // 2 - 1)
        def _():
            issue(i0 + 2, 0)

        finish(i0 + 1, 1)
        return carry

    lax.fori_loop(0, NCHUNK // 2, super_iter, 0)

    cw1.wait()
    cw2.wait()

    sbase = wid * SEEDS_PER_W
    for j in range(SEEDS_PER_W // CH):
        soff = sbase + j * CH
        ra, _, sg = rows[j % 2]
        pltpu.sync_copy(seeds_hbm.at[pl.ds(soff, CH)], sidx_all.at[pl.ds(0, CH)])
        pltpu.async_copy(semb_hbm.at[sidx_all.at[pl.ds(0, CH)]], ra, sg).wait()
        pltpu.sync_copy(ra, self_out.at[pl.ds(soff, CH)])


def _sc_gather(ev, seeds, esrc, edst, ets, ew, psrc, pdst, semb):
    mesh = plsc.VectorSubcoreMesh(core_axis_name="c", subcore_axis_name="s")
    fn = pl.kernel(
        _sc_body,
        out_type=[
            jax.ShapeDtypeStruct((EV, H), jnp.float32),
            jax.ShapeDtypeStruct((EV,), jnp.float32),
            jax.ShapeDtypeStruct((EV,), jnp.float32),
            jax.ShapeDtypeStruct((B, D), jnp.float32),
        ],
        mesh=mesh,
        scratch_types=[
            pltpu.VMEM((EV_PER_W,), jnp.int32),
            pltpu.VMEM((EV_PER_W,), jnp.int32),
            pltpu.VMEM((EV_PER_W,), jnp.int32),
            pltpu.VMEM((EV_PER_W,), jnp.float32),
            pltpu.VMEM((EV_PER_W,), jnp.float32),
            pltpu.VMEM((CH, H), jnp.float32),
            pltpu.VMEM((CH, H), jnp.float32),
            pltpu.VMEM((CH, H), jnp.float32),
            pltpu.VMEM((CH, H), jnp.float32),
            pltpu.SemaphoreType.DMA,
            pltpu.SemaphoreType.DMA,
            pltpu.SemaphoreType.DMA,
            pltpu.SemaphoreType.DMA,
        ],
    )
    return fn(ev, seeds, esrc, edst, ets, ew, psrc, pdst, semb)


def _finish_body(g_ref, ts_ref, w_ref, self_ref, w1t_ref, b1_ref, w2_ref,
                 b2_ref, wself_ref, wneigh_ref, out_ref):
    dn = (((1,), (1,)), ((), ()))
    ts_t = jnp.transpose(ts_ref[...]) * (1.0 / TS_RANGE)
    wv_t = jnp.log1p(jnp.transpose(w_ref[...]))
    nsub = ts_t.shape[1]
    w1a = w1t_ref[0:1, :]
    w1b = w1t_ref[1:2, :]
    b1v = b1_ref[...]
    pieces = [
        jnp.maximum(ts_t[:, r:r + 1] * w1a + wv_t[:, r:r + 1] * w1b + b1v,
                    0.0)
        for r in range(nsub)
    ]
    h1 = jnp.concatenate(pieces, axis=0)
    mlp = lax.dot_general(h1, w2_ref[...], dn,
                          preferred_element_type=jnp.float32) + b2_ref[...]
    evh = jnp.maximum(mlp + g_ref[...], 0.0)
    sb = evh.shape[0] // FANOUT
    neigh = jnp.mean(evh.reshape(sb, FANOUT, H), axis=1)
    out = lax.dot_general(self_ref[...], wself_ref[...], dn,
                          preferred_element_type=jnp.float32)
    out += lax.dot_general(neigh, wneigh_ref[...], dn,
                           preferred_element_type=jnp.float32)
    out_ref[...] = jnp.maximum(out, 0.0)


def _finish(g, ts, w, self_rows, w1t, b1, w2, b2, wself, wneigh):
    sb = 256
    eb = sb * FANOUT
    nblk = B // sb
    return pl.pallas_call(
        _finish_body,
        grid=(nblk,),
        in_specs=[
            pl.BlockSpec((eb, H), lambda i: (i, 0)),
            pl.BlockSpec((eb // 128, 128), lambda i: (i, 0)),
            pl.BlockSpec((eb // 128, 128), lambda i: (i, 0)),
            pl.BlockSpec((sb, D), lambda i: (i, 0)),
            pl.BlockSpec((2, H), lambda i: (0, 0)),
            pl.BlockSpec((1, H), lambda i: (0, 0)),
            pl.BlockSpec((H, H), lambda i: (0, 0)),
            pl.BlockSpec((1, H), lambda i: (0, 0)),
            pl.BlockSpec((H, D), lambda i: (0, 0)),
            pl.BlockSpec((H, H), lambda i: (0, 0)),
        ],
        out_specs=pl.BlockSpec((sb, H), lambda i: (i, 0)),
        out_shape=jax.ShapeDtypeStruct((B, H), jnp.float32),
    )(g, ts, w, self_rows, w1t, b1, w2, b2, wself, wneigh)


def kernel(seeds, nbr_ev, event_src, event_dst, event_ts_s, event_w,
           src_emb, dst_emb, W1, b1, W2, b2,
           W_event_src, W_event_dst, W_self, W_neigh):
    ev = nbr_ev.reshape(EV).astype(jnp.int32)
    seeds32 = seeds.astype(jnp.int32)
    psrc, pdst = _precompute_tables(src_emb, dst_emb, W_event_src,
                                    W_event_dst)
    g, ts_g, w_g, self_rows = _sc_gather(
        ev, seeds32, event_src, event_dst, event_ts_s, event_w,
        psrc, pdst, src_emb)
    out = _finish(g, ts_g.reshape(EV // 128, 128), w_g.reshape(EV // 128, 128),
                  self_rows,
                  W1.T, b1.reshape(1, H), W2, b2.reshape(1, H),
                  W_self, W_neigh)
    return out
